# baseline (speedup 1.0000x reference)
import functools

import jax
import jax.numpy as jnp
from jax import lax
from jax.experimental import pallas as pl
from jax.experimental.pallas import tpu as pltpu
from jax.experimental.pallas import tpu_sc as plsc

_B = 16384
_DIM = 32
_NC = 2
_NS = 16
_NW = _NC * _NS
_BPW = _B // _NW
_CH = 128
_NCH = _BPW // _CH


def _body(user_idx, item_idx, time_idx, su, si, st,
          ut, it, tt, out,
          uiv, iiv, tiv, suv, siv, stv, rows, sem):
    wid = lax.axis_index("s") * _NC + lax.axis_index("c")
    base = wid * _BPW

    pltpu.sync_copy(user_idx.at[pl.ds(base, _BPW)], uiv)
    pltpu.sync_copy(item_idx.at[pl.ds(base, _BPW)], iiv)
    pltpu.sync_copy(time_idx.at[pl.ds(base, _BPW)], tiv)
    for j in range(_NCH):
        pltpu.sync_copy(su.at[pl.ds(base + j * _CH, _CH)], suv.at[j])
        pltpu.sync_copy(si.at[pl.ds(base + j * _CH, _CH)], siv.at[j])
        pltpu.sync_copy(st.at[pl.ds(base + j * _CH, _CH)], stv.at[j])

    waits = []
    for j in range(_NCH):
        waits.append(pltpu.async_copy(
            ut.at[uiv.at[pl.ds(j * _CH, _CH)]], rows.at[0, j], sem))
        waits.append(pltpu.async_copy(
            it.at[iiv.at[pl.ds(j * _CH, _CH)]], rows.at[1, j], sem))
        waits.append(pltpu.async_copy(
            tt.at[tiv.at[pl.ds(j * _CH, _CH)]], rows.at[2, j], sem))
    for w in waits:
        w.wait()

    for j in range(_NCH):
        pltpu.sync_copy(rows.at[0, j], out.at[suv.at[j]])
        pltpu.sync_copy(rows.at[1, j], out.at[siv.at[j]])
        pltpu.sync_copy(rows.at[2, j], out.at[stv.at[j]])


_emb_call = functools.partial(
    pl.kernel,
    out_type=jax.ShapeDtypeStruct((3 * _B, _DIM), jnp.float32),
    mesh=plsc.VectorSubcoreMesh(core_axis_name="c", subcore_axis_name="s"),
    scratch_types=[
        pltpu.VMEM((_BPW,), jnp.int32),
        pltpu.VMEM((_BPW,), jnp.int32),
        pltpu.VMEM((_BPW,), jnp.int32),
        pltpu.VMEM((_NCH, _CH), jnp.int32),
        pltpu.VMEM((_NCH, _CH), jnp.int32),
        pltpu.VMEM((_NCH, _CH), jnp.int32),
        pltpu.VMEM((3, _NCH, _CH, _DIM), jnp.float32),
        pltpu.SemaphoreType.DMA,
    ],
    compiler_params=pltpu.CompilerParams(use_tc_tiling_on_sc=False),
)(_body)


@jax.jit
def kernel(user_idx, item_idx, time_idx, user_table, item_table, time_table):
    scatter_base = jnp.arange(_B, dtype=jnp.int32) * 3
    out = _emb_call(user_idx, item_idx, time_idx,
                    scatter_base, scatter_base + 1, scatter_base + 2,
                    user_table, item_table, time_table)
    return out.reshape(_B, 3 * _DIM)
